# baseline (speedup 1.0000x reference)
import jax
import jax.numpy as jnp
from jax.experimental import pallas as pl
from jax.experimental.pallas import tpu as pltpu


def _body(rhs_ref, k_ref, d_ref, i_ref):
    step = pl.program_id(0)
    bk = k_ref.shape[0]

    k = k_ref[:, :]
    lhs = jnp.concatenate([k, k * k], axis=1)
    dist = jax.lax.dot_general(
        lhs, rhs_ref[:, :], (((1,), (0,)), ((), ())),
        preferred_element_type=jnp.float32)
    dist_t = dist.T

    cols = jax.lax.broadcasted_iota(jnp.int32, dist_t.shape, 1)
    m1 = jnp.min(dist_t, axis=1, keepdims=True)
    i1 = jnp.min(jnp.where(dist_t == m1, cols, bk),
                 axis=1, keepdims=True) + step * bk

    @pl.when(step == 0)
    def _init():
        d_ref[:, :] = jnp.full(d_ref.shape, jnp.inf, jnp.float32)
        i_ref[:, :] = jnp.zeros(i_ref.shape, jnp.int32)

    b1 = m1 < d_ref[:, :]
    i_ref[:, :] = jnp.where(b1, i1, i_ref[:, :])
    d_ref[:, :] = jnp.where(b1, m1, d_ref[:, :])


def kernel(queries, keys):
    q_n, dim = queries.shape
    n_keys = keys.shape[0]
    bk = 20000
    grid = (n_keys // bk,)

    rhs = jnp.concatenate(
        [-2.0 * queries.T, jnp.ones((dim, q_n), jnp.float32)], axis=0)

    d_out, i_out = pl.pallas_call(
        _body,
        grid=grid,
        in_specs=[
            pl.BlockSpec((2 * dim, q_n), lambda i: (0, 0)),
            pl.BlockSpec((bk, dim), lambda i: (i, 0)),
        ],
        out_specs=[
            pl.BlockSpec((q_n, 1), lambda i: (0, 0)),
            pl.BlockSpec((q_n, 1), lambda i: (0, 0)),
        ],
        out_shape=[
            jax.ShapeDtypeStruct((q_n, 1), jnp.float32),
            jax.ShapeDtypeStruct((q_n, 1), jnp.int32),
        ],
        compiler_params=pltpu.CompilerParams(
            dimension_semantics=("arbitrary",)),
    )(rhs, keys)

    qsq = jnp.sum(queries * queries, axis=1, keepdims=True)
    return (d_out + qsq, i_out)
